# baseline (speedup 1.0000x reference)
import functools

import jax
import jax.numpy as jnp
from jax import lax
from jax.experimental import pallas as pl
from jax.experimental.pallas import tpu as pltpu
from jax.experimental.pallas import tpu_sc as plsc

_B, _S, _D, _E = 2, 2048, 768, 8
_N = _B * _S
_TM = 1024

_NW = 32
_TPW = _N // _NW
_L = 16


def _gate_body(x_ref, gw_ref, gb_ref, logits_ref, xb16_ref):
    xb16 = x_ref[...].astype(jnp.bfloat16)
    xb16_ref[...] = xb16
    logits_ref[...] = jax.lax.dot_general(
        gw_ref[...].astype(jnp.bfloat16), xb16, (((1,), (1,)), ((), ())),
        preferred_element_type=jnp.float32,
    ) + gb_ref[...]


def _sc_topk_body(logits_hbm, w_hbm, lv, wv):
    wid = lax.axis_index("s") * 2 + lax.axis_index("c")
    base = wid * _TPW
    for e in range(_E):
        pltpu.sync_copy(logits_hbm.at[e, pl.ds(base, _TPW)],
                        lv.at[pl.ds(e * _TPW, _TPW)])
    for c in range(_TPW // _L):
        cols = [lv[pl.ds(e * _TPW + c * _L, _L)] for e in range(_E)]
        m1 = cols[0]
        i1 = jnp.zeros((_L,), jnp.int32)
        for e in range(1, _E):
            b = cols[e] > m1
            m1 = jnp.where(b, cols[e], m1)
            i1 = jnp.where(b, e, i1)
        m2 = jnp.full((_L,), -jnp.inf, jnp.float32)
        i2 = jnp.zeros((_L,), jnp.int32)
        for e in range(_E):
            cnd = jnp.logical_and(i1 != e, cols[e] > m2)
            m2 = jnp.where(cnd, cols[e], m2)
            i2 = jnp.where(cnd, e, i2)
        for e in range(_E):
            sel = jnp.logical_or(i1 == e, i2 == e)
            wv[pl.ds(e * _TPW + c * _L, _L)] = jnp.where(sel, cols[e], 0.0)
    for e in range(_E):
        pltpu.sync_copy(wv.at[pl.ds(e * _TPW, _TPW)],
                        w_hbm.at[e, pl.ds(base, _TPW)])


def _dispatch_body(xb16_ref, wt_ref, ew_ref, eb_ref, out_ref, wf_scr):
    t = pl.program_id(0)

    @pl.when(t == 0)
    def _cvt():
        wf_scr[...] = ew_ref[...].astype(jnp.bfloat16)

    xb16 = xb16_ref[...]
    wt16 = wt_ref[...].astype(jnp.bfloat16)
    acc = jax.lax.dot_general(
        wt16, eb_ref[...].astype(jnp.bfloat16), (((0,), (0,)), ((), ())),
        preferred_element_type=jnp.float32)
    w16 = jax.lax.dot_general(
        wt16, jnp.eye(_E, dtype=jnp.bfloat16), (((0,), (0,)), ((), ())),
        preferred_element_type=jnp.float32).astype(jnp.bfloat16)
    for e in range(_E):
        xs = xb16 * w16[:, e:e + 1]
        acc = acc + jax.lax.dot_general(
            xs, wf_scr[e * _D:(e + 1) * _D, :], (((1,), (1,)), ((), ())),
            preferred_element_type=jnp.float32)
    out_ref[...] = acc


def kernel(x, router_mask, gate_w, gate_b, expert_w, expert_b):
    xf = x.reshape(_N, _D)

    logits_t, xb16 = pl.pallas_call(
        _gate_body,
        grid=(_N // _TM,),
        in_specs=[
            pl.BlockSpec((_TM, _D), lambda t: (t, 0)),
            pl.BlockSpec((_E, _D), lambda t: (0, 0)),
            pl.BlockSpec((_E, 1), lambda t: (0, 0)),
        ],
        out_specs=(
            pl.BlockSpec((_E, _TM), lambda t: (0, t)),
            pl.BlockSpec((_TM, _D), lambda t: (t, 0)),
        ),
        out_shape=(
            jax.ShapeDtypeStruct((_E, _N), jnp.float32),
            jax.ShapeDtypeStruct((_N, _D), jnp.bfloat16),
        ),
        compiler_params=pltpu.CompilerParams(
            dimension_semantics=("arbitrary",)),
    )(xf, gate_w, gate_b.reshape(_E, 1))

    sc_topk = functools.partial(
        pl.kernel,
        mesh=plsc.VectorSubcoreMesh(core_axis_name="c", subcore_axis_name="s"),
        out_type=jax.ShapeDtypeStruct((_E, _N), jnp.float32),
        scratch_types=[
            pltpu.VMEM((_E * _TPW,), jnp.float32),
            pltpu.VMEM((_E * _TPW,), jnp.float32),
        ],
    )(_sc_topk_body)
    w_t = sc_topk(logits_t)

    out = pl.pallas_call(
        _dispatch_body,
        grid=(_N // _TM,),
        in_specs=[
            pl.BlockSpec((_TM, _D), lambda t: (t, 0)),
            pl.BlockSpec((_E, _TM), lambda t: (0, t)),
            pl.BlockSpec((_E * _D, _D), lambda t: (0, 0)),
            pl.BlockSpec((_E, _D), lambda t: (0, 0)),
        ],
        out_specs=pl.BlockSpec((_TM, _D), lambda t: (t, 0)),
        out_shape=jax.ShapeDtypeStruct((_N, _D), jnp.float32),
        scratch_shapes=[pltpu.VMEM((_E * _D, _D), jnp.bfloat16)],
        compiler_params=pltpu.CompilerParams(
            dimension_semantics=("arbitrary",)),
    )(xb16, w_t, expert_w.reshape(_E * _D, _D), expert_b)
    return out.reshape(_B, _S, _D)


# speedup vs baseline: 1.0452x; 1.0452x over previous
import functools

import jax
import jax.numpy as jnp
from jax import lax
from jax.experimental import pallas as pl
from jax.experimental.pallas import tpu as pltpu
from jax.experimental.pallas import tpu_sc as plsc

_B, _S, _D, _E = 2, 2048, 768, 8
_N = _B * _S
_TM = 1024

_NW = 32
_TPW = _N // _NW
_L = 16


def _gate_body(x_ref, gw_ref, gb_ref, logits_ref, xb16_ref):
    xb16 = x_ref[...].astype(jnp.bfloat16)
    xb16_ref[...] = xb16
    logits_ref[...] = jax.lax.dot_general(
        gw_ref[...].astype(jnp.bfloat16), xb16, (((1,), (1,)), ((), ())),
        preferred_element_type=jnp.float32,
    ) + gb_ref[...]


def _sc_topk_body(logits_hbm, w_hbm, lv, wv, sem):
    wid = lax.axis_index("s") * 2 + lax.axis_index("c")
    base = wid * _TPW
    cps = [pltpu.make_async_copy(logits_hbm.at[e, pl.ds(base, _TPW)],
                                 lv.at[pl.ds(e * _TPW, _TPW)], sem)
           for e in range(_E)]
    for cp in cps:
        cp.start()
    for cp in cps:
        cp.wait()
    for c in range(_TPW // _L):
        cols = [lv[pl.ds(e * _TPW + c * _L, _L)] for e in range(_E)]
        m1 = cols[0]
        i1 = jnp.zeros((_L,), jnp.int32)
        for e in range(1, _E):
            b = cols[e] > m1
            m1 = jnp.where(b, cols[e], m1)
            i1 = jnp.where(b, e, i1)
        m2 = jnp.full((_L,), -jnp.inf, jnp.float32)
        i2 = jnp.zeros((_L,), jnp.int32)
        for e in range(_E):
            cnd = jnp.logical_and(i1 != e, cols[e] > m2)
            m2 = jnp.where(cnd, cols[e], m2)
            i2 = jnp.where(cnd, e, i2)
        for e in range(_E):
            sel = jnp.logical_or(i1 == e, i2 == e)
            wv[pl.ds(e * _TPW + c * _L, _L)] = jnp.where(sel, cols[e], 0.0)
    ops = [pltpu.make_async_copy(wv.at[pl.ds(e * _TPW, _TPW)],
                                 w_hbm.at[e, pl.ds(base, _TPW)], sem)
           for e in range(_E)]
    for cp in ops:
        cp.start()
    for cp in ops:
        cp.wait()


def _dispatch_body(xb16_ref, wt_ref, ew_ref, eb_ref, out_ref, wf_scr):
    t = pl.program_id(0)

    @pl.when(t == 0)
    def _cvt():
        wf_scr[...] = ew_ref[...].astype(jnp.bfloat16)

    xb16 = xb16_ref[...]
    wt16 = wt_ref[...].astype(jnp.bfloat16)
    acc = jax.lax.dot_general(
        wt16, eb_ref[...].astype(jnp.bfloat16), (((0,), (0,)), ((), ())),
        preferred_element_type=jnp.float32)
    w16 = jax.lax.dot_general(
        wt16, jnp.eye(_E, dtype=jnp.bfloat16), (((0,), (0,)), ((), ())),
        preferred_element_type=jnp.float32).astype(jnp.bfloat16)
    for e in range(_E):
        xs = xb16 * w16[:, e:e + 1]
        acc = acc + jax.lax.dot_general(
            xs, wf_scr[e * _D:(e + 1) * _D, :], (((1,), (1,)), ((), ())),
            preferred_element_type=jnp.float32)
    out_ref[...] = acc


def kernel(x, router_mask, gate_w, gate_b, expert_w, expert_b):
    xf = x.reshape(_N, _D)

    logits_t, xb16 = pl.pallas_call(
        _gate_body,
        grid=(_N // _TM,),
        in_specs=[
            pl.BlockSpec((_TM, _D), lambda t: (t, 0)),
            pl.BlockSpec((_E, _D), lambda t: (0, 0)),
            pl.BlockSpec((_E, 1), lambda t: (0, 0)),
        ],
        out_specs=(
            pl.BlockSpec((_E, _TM), lambda t: (0, t)),
            pl.BlockSpec((_TM, _D), lambda t: (t, 0)),
        ),
        out_shape=(
            jax.ShapeDtypeStruct((_E, _N), jnp.float32),
            jax.ShapeDtypeStruct((_N, _D), jnp.bfloat16),
        ),
        compiler_params=pltpu.CompilerParams(
            dimension_semantics=("arbitrary",)),
    )(xf, gate_w, gate_b.reshape(_E, 1))

    sc_topk = functools.partial(
        pl.kernel,
        mesh=plsc.VectorSubcoreMesh(core_axis_name="c", subcore_axis_name="s"),
        out_type=jax.ShapeDtypeStruct((_E, _N), jnp.float32),
        scratch_types=[
            pltpu.VMEM((_E * _TPW,), jnp.float32),
            pltpu.VMEM((_E * _TPW,), jnp.float32),
            pltpu.SemaphoreType.DMA,
        ],
    )(_sc_topk_body)
    w_t = sc_topk(logits_t)

    out = pl.pallas_call(
        _dispatch_body,
        grid=(_N // _TM,),
        in_specs=[
            pl.BlockSpec((_TM, _D), lambda t: (t, 0)),
            pl.BlockSpec((_E, _TM), lambda t: (0, t)),
            pl.BlockSpec((_E * _D, _D), lambda t: (0, 0)),
            pl.BlockSpec((_E, _D), lambda t: (0, 0)),
        ],
        out_specs=pl.BlockSpec((_TM, _D), lambda t: (t, 0)),
        out_shape=jax.ShapeDtypeStruct((_N, _D), jnp.float32),
        scratch_shapes=[pltpu.VMEM((_E * _D, _D), jnp.bfloat16)],
        compiler_params=pltpu.CompilerParams(
            dimension_semantics=("arbitrary",)),
    )(xb16, w_t, expert_w.reshape(_E * _D, _D), expert_b)
    return out.reshape(_B, _S, _D)
